# baseline (speedup 1.0000x reference)
import functools

import jax
import jax.numpy as jnp
from jax import lax
from jax.experimental import pallas as pl
from jax.experimental.pallas import tpu as pltpu
from jax.experimental.pallas import tpu_sc as plsc

B, S, V, D = 16384, 200, 100, 10
VPAD = 104
REP = 16

_info = plsc.get_sparse_core_info()
NC, NS, L = _info.num_cores, _info.num_subcores, _info.num_lanes
NW = NC * NS
BW = B // NW
ST = S // 8
KG = BW // L


def _sc_body(x_hbm, tab_hbm, out_hbm, tab_v, x_v0, x_v1, out_v0, out_v1,
             sem0, sem1, semx0, semx1):
    wid = lax.axis_index("s") * NC + lax.axis_index("c")
    b0 = wid * BW

    pltpu.sync_copy(tab_hbm, tab_v)
    iota = lax.iota(jnp.int32, L)
    zeros = jnp.full((L,), 0.0, jnp.float32)
    for d in range(D):
        tab_v[pl.ds(d * VPAD * REP, L)] = zeros

    dbase = [jnp.full((L,), d * VPAD * REP, jnp.int32) for d in range(D)]

    def x_slice(st):
        return x_hbm.at[pl.ds(st * 8, 8), pl.ds(b0, BW)]

    def out_slice(st):
        return out_hbm.at[:, pl.ds(st * 8, 8), pl.ds(b0, BW)]

    def compute_tile(x_v, out_v):
        for s_in in range(8):
            @plsc.parallel_loop(0, KG, unroll=2)
            def k_body(k):
                x16 = x_v[s_in, pl.ds(k * L, L)]
                lane_idx = (x16 << 4) + iota
                vals = [plsc.load_gather(tab_v, [dbase[d] + lane_idx])
                        for d in range(D)]
                for d in range(D):
                    out_v[d, s_in, pl.ds(k * L, L)] = vals[d]

    pltpu.async_copy(x_slice(0), x_v0, semx0)
    pltpu.async_copy(x_slice(1), x_v1, semx1)

    pltpu.make_async_copy(x_slice(0), x_v0, semx0).wait()
    compute_tile(x_v0, out_v0)
    pltpu.async_copy(out_v0, out_slice(0), sem0)
    pltpu.async_copy(x_slice(2), x_v0, semx0)

    pltpu.make_async_copy(x_slice(1), x_v1, semx1).wait()
    compute_tile(x_v1, out_v1)
    pltpu.async_copy(out_v1, out_slice(1), sem1)
    pltpu.async_copy(x_slice(3), x_v1, semx1)

    def pair_body(g, carry):
        st = 2 + 2 * g
        pltpu.make_async_copy(out_v0, out_slice(st), sem0).wait()
        pltpu.make_async_copy(x_slice(st), x_v0, semx0).wait()
        compute_tile(x_v0, out_v0)
        pltpu.async_copy(out_v0, out_slice(st), sem0)
        pltpu.async_copy(x_slice(st + 2), x_v0, semx0)

        pltpu.make_async_copy(out_v1, out_slice(st + 1), sem1).wait()
        pltpu.make_async_copy(x_slice(st + 1), x_v1, semx1).wait()
        compute_tile(x_v1, out_v1)
        pltpu.async_copy(out_v1, out_slice(st + 1), sem1)

        @pl.when(st + 3 < ST)
        def _():
            pltpu.async_copy(x_slice(st + 3), x_v1, semx1)

        return carry

    lax.fori_loop(0, (ST - 3) // 2, pair_body, 0)

    pltpu.make_async_copy(out_v0, out_slice(ST - 1), sem0).wait()
    pltpu.make_async_copy(x_slice(ST - 1), x_v0, semx0).wait()
    compute_tile(x_v0, out_v0)
    pltpu.async_copy(out_v0, out_slice(ST - 1), sem0)
    pltpu.make_async_copy(out_v0, out_slice(ST - 1), sem0).wait()
    pltpu.make_async_copy(out_v1, out_slice(ST - 2), sem1).wait()


@jax.jit
def kernel(x, table):
    tab_t = jnp.zeros((D, VPAD), jnp.float32).at[:, :V].set(table.T)
    tab_rep = jnp.broadcast_to(tab_t[:, :, None], (D, VPAD, REP)).reshape(-1)
    xt = x.T.astype(jnp.int32)

    mesh = plsc.VectorSubcoreMesh(core_axis_name="c", subcore_axis_name="s")
    run = functools.partial(
        pl.kernel,
        mesh=mesh,
        out_type=jax.ShapeDtypeStruct((D, S, B), jnp.float32),
        scratch_types=[
            pltpu.VMEM((D * VPAD * REP,), jnp.float32),
            pltpu.VMEM((8, BW), jnp.int32),
            pltpu.VMEM((8, BW), jnp.int32),
            pltpu.VMEM((D, 8, BW), jnp.float32),
            pltpu.VMEM((D, 8, BW), jnp.float32),
            pltpu.SemaphoreType.DMA,
            pltpu.SemaphoreType.DMA,
            pltpu.SemaphoreType.DMA,
            pltpu.SemaphoreType.DMA,
        ],
        compiler_params=pltpu.CompilerParams(
            needs_layout_passes=False,
            use_tc_tiling_on_sc=True,
        ),
    )(_sc_body)
    out_t = run(xt, tab_rep)
    return out_t.transpose(2, 1, 0)


# speedup vs baseline: 1.2178x; 1.2178x over previous
import functools

import jax
import jax.numpy as jnp
from jax import lax
from jax.experimental import pallas as pl
from jax.experimental.pallas import tpu as pltpu
from jax.experimental.pallas import tpu_sc as plsc

B, S, V, D = 16384, 200, 100, 10
VPAD = 104
REP = 16

_info = plsc.get_sparse_core_info()
NC, NS, L = _info.num_cores, _info.num_subcores, _info.num_lanes
NW = NC * NS
BW = B // NW
ST = S // 8
KG = BW // L


def _sc_body(x_hbm, tab_hbm, out_hbm, tab_v, x_v0, x_v1, out_v0, out_v1,
             sem0, sem1, semx0, semx1):
    wid = lax.axis_index("s") * NC + lax.axis_index("c")
    b0 = wid * BW

    pltpu.sync_copy(tab_hbm, tab_v)
    iota = lax.iota(jnp.int32, L)
    zeros = jnp.full((L,), 0.0, jnp.float32)
    for d in range(D):
        tab_v[pl.ds(d * VPAD * REP, L)] = zeros

    dbase = [jnp.full((L,), d * VPAD * REP, jnp.int32) for d in range(D)]

    def x_slice(st):
        return x_hbm.at[pl.ds(st * 8, 8), pl.ds(b0, BW)]

    def out_slice(st):
        return out_hbm.at[:, pl.ds(st * 8, 8), pl.ds(b0, BW)]

    def compute_tile(x_v, out_v):
        for s_in in range(8):
            @plsc.parallel_loop(0, KG, unroll=1)
            def k_body(k):
                x16 = x_v[s_in, pl.ds(k * L, L)]
                lane_idx = (x16 << 4) + iota
                vals = [plsc.load_gather(tab_v, [dbase[d] + lane_idx])
                        for d in range(D)]
                for d in range(D):
                    out_v[d, s_in, pl.ds(k * L, L)] = vals[d]

    pltpu.async_copy(x_slice(0), x_v0, semx0)
    pltpu.async_copy(x_slice(1), x_v1, semx1)

    pltpu.make_async_copy(x_slice(0), x_v0, semx0).wait()
    compute_tile(x_v0, out_v0)
    pltpu.async_copy(out_v0, out_slice(0), sem0)
    pltpu.async_copy(x_slice(2), x_v0, semx0)

    pltpu.make_async_copy(x_slice(1), x_v1, semx1).wait()
    compute_tile(x_v1, out_v1)
    pltpu.async_copy(out_v1, out_slice(1), sem1)
    pltpu.async_copy(x_slice(3), x_v1, semx1)

    def pair_body(g, carry):
        st = 2 + 2 * g
        pltpu.make_async_copy(out_v0, out_slice(st), sem0).wait()
        pltpu.make_async_copy(x_slice(st), x_v0, semx0).wait()
        compute_tile(x_v0, out_v0)
        pltpu.async_copy(out_v0, out_slice(st), sem0)
        pltpu.async_copy(x_slice(st + 2), x_v0, semx0)

        pltpu.make_async_copy(out_v1, out_slice(st + 1), sem1).wait()
        pltpu.make_async_copy(x_slice(st + 1), x_v1, semx1).wait()
        compute_tile(x_v1, out_v1)
        pltpu.async_copy(out_v1, out_slice(st + 1), sem1)

        @pl.when(st + 3 < ST)
        def _():
            pltpu.async_copy(x_slice(st + 3), x_v1, semx1)

        return carry

    lax.fori_loop(0, (ST - 3) // 2, pair_body, 0)

    pltpu.make_async_copy(out_v0, out_slice(ST - 1), sem0).wait()
    pltpu.make_async_copy(x_slice(ST - 1), x_v0, semx0).wait()
    compute_tile(x_v0, out_v0)
    pltpu.async_copy(out_v0, out_slice(ST - 1), sem0)
    pltpu.make_async_copy(out_v0, out_slice(ST - 1), sem0).wait()
    pltpu.make_async_copy(out_v1, out_slice(ST - 2), sem1).wait()


@jax.jit
def kernel(x, table):
    tab_t = jnp.zeros((D, VPAD), jnp.float32).at[:, :V].set(table.T)
    tab_rep = jnp.broadcast_to(tab_t[:, :, None], (D, VPAD, REP)).reshape(-1)
    xt = x.T.astype(jnp.int32)

    mesh = plsc.VectorSubcoreMesh(core_axis_name="c", subcore_axis_name="s")
    run = functools.partial(
        pl.kernel,
        mesh=mesh,
        out_type=jax.ShapeDtypeStruct((D, S, B), jnp.float32),
        scratch_types=[
            pltpu.VMEM((D * VPAD * REP,), jnp.float32),
            pltpu.VMEM((8, BW), jnp.int32),
            pltpu.VMEM((8, BW), jnp.int32),
            pltpu.VMEM((D, 8, BW), jnp.float32),
            pltpu.VMEM((D, 8, BW), jnp.float32),
            pltpu.SemaphoreType.DMA,
            pltpu.SemaphoreType.DMA,
            pltpu.SemaphoreType.DMA,
            pltpu.SemaphoreType.DMA,
        ],
        compiler_params=pltpu.CompilerParams(
            needs_layout_passes=False,
            use_tc_tiling_on_sc=True,
        ),
    )(_sc_body)
    out_t = run(xt, tab_rep)
    return out_t.transpose(2, 1, 0)


# speedup vs baseline: 1.2848x; 1.0550x over previous
import functools

import jax
import jax.numpy as jnp
from jax import lax
from jax.experimental import pallas as pl
from jax.experimental.pallas import tpu as pltpu
from jax.experimental.pallas import tpu_sc as plsc

B, S, V, D = 16384, 200, 100, 10
VPAD = 104
REP = 16

_info = plsc.get_sparse_core_info()
NC, NS, L = _info.num_cores, _info.num_subcores, _info.num_lanes
NW = NC * NS
BW = B // NW
ST = S // 8
KG = BW // L


def _sc_body(x_hbm, tab_hbm, out_hbm, tab_v, x_v0, x_v1, out_v0, out_v1,
             sem0, sem1, semx0, semx1):
    wid = lax.axis_index("s") * NC + lax.axis_index("c")
    b0 = wid * BW

    pltpu.sync_copy(tab_hbm, tab_v)
    iota = lax.iota(jnp.int32, L)
    zeros = jnp.full((L,), 0.0, jnp.float32)
    for d in range(D):
        tab_v[pl.ds(d * VPAD * REP, L)] = zeros

    dbase = [jnp.full((L,), d * VPAD * REP, jnp.int32) for d in range(D)]

    def x_slice(st):
        return x_hbm.at[pl.ds(st * 8, 8), pl.ds(b0, BW)]

    def out_slice(st):
        return out_hbm.at[:, pl.ds(st * 8, 8), pl.ds(b0, BW)]

    def compute_tile(x_v, out_v):
        @plsc.parallel_loop(0, 8 * KG, unroll=1)
        def k_body(k):
            s_in = k >> 5
            kk = k & (KG - 1)
            x16 = x_v[s_in, pl.ds(kk * L, L)]
            lane_idx = (x16 << 4) + iota
            vals = [plsc.load_gather(tab_v, [dbase[d] + lane_idx])
                    for d in range(D)]
            for d in range(D):
                out_v[d, s_in, pl.ds(kk * L, L)] = vals[d]

    pltpu.async_copy(x_slice(0), x_v0, semx0)
    pltpu.async_copy(x_slice(1), x_v1, semx1)

    pltpu.make_async_copy(x_slice(0), x_v0, semx0).wait()
    compute_tile(x_v0, out_v0)
    pltpu.async_copy(out_v0, out_slice(0), sem0)
    pltpu.async_copy(x_slice(2), x_v0, semx0)

    pltpu.make_async_copy(x_slice(1), x_v1, semx1).wait()
    compute_tile(x_v1, out_v1)
    pltpu.async_copy(out_v1, out_slice(1), sem1)
    pltpu.async_copy(x_slice(3), x_v1, semx1)

    def pair_body(g, carry):
        st = 2 + 2 * g
        pltpu.make_async_copy(out_v0, out_slice(st), sem0).wait()
        pltpu.make_async_copy(x_slice(st), x_v0, semx0).wait()
        compute_tile(x_v0, out_v0)
        pltpu.async_copy(out_v0, out_slice(st), sem0)
        pltpu.async_copy(x_slice(st + 2), x_v0, semx0)

        pltpu.make_async_copy(out_v1, out_slice(st + 1), sem1).wait()
        pltpu.make_async_copy(x_slice(st + 1), x_v1, semx1).wait()
        compute_tile(x_v1, out_v1)
        pltpu.async_copy(out_v1, out_slice(st + 1), sem1)

        @pl.when(st + 3 < ST)
        def _():
            pltpu.async_copy(x_slice(st + 3), x_v1, semx1)

        return carry

    lax.fori_loop(0, (ST - 3) // 2, pair_body, 0)

    pltpu.make_async_copy(out_v0, out_slice(ST - 1), sem0).wait()
    pltpu.make_async_copy(x_slice(ST - 1), x_v0, semx0).wait()
    compute_tile(x_v0, out_v0)
    pltpu.async_copy(out_v0, out_slice(ST - 1), sem0)
    pltpu.make_async_copy(out_v0, out_slice(ST - 1), sem0).wait()
    pltpu.make_async_copy(out_v1, out_slice(ST - 2), sem1).wait()


@jax.jit
def kernel(x, table):
    tab_t = jnp.zeros((D, VPAD), jnp.float32).at[:, :V].set(table.T)
    tab_rep = jnp.broadcast_to(tab_t[:, :, None], (D, VPAD, REP)).reshape(-1)
    xt = x.T.astype(jnp.int32)

    mesh = plsc.VectorSubcoreMesh(core_axis_name="c", subcore_axis_name="s")
    run = functools.partial(
        pl.kernel,
        mesh=mesh,
        out_type=jax.ShapeDtypeStruct((D, S, B), jnp.float32),
        scratch_types=[
            pltpu.VMEM((D * VPAD * REP,), jnp.float32),
            pltpu.VMEM((8, BW), jnp.int32),
            pltpu.VMEM((8, BW), jnp.int32),
            pltpu.VMEM((D, 8, BW), jnp.float32),
            pltpu.VMEM((D, 8, BW), jnp.float32),
            pltpu.SemaphoreType.DMA,
            pltpu.SemaphoreType.DMA,
            pltpu.SemaphoreType.DMA,
            pltpu.SemaphoreType.DMA,
        ],
        compiler_params=pltpu.CompilerParams(
            needs_layout_passes=False,
            use_tc_tiling_on_sc=True,
        ),
    )(_sc_body)
    out_t = run(xt, tab_rep)
    return out_t.transpose(2, 1, 0)


# speedup vs baseline: 1.2963x; 1.0089x over previous
import functools

import jax
import jax.numpy as jnp
from jax import lax
from jax.experimental import pallas as pl
from jax.experimental.pallas import tpu as pltpu
from jax.experimental.pallas import tpu_sc as plsc

B, S, V, D = 16384, 200, 100, 10
VPAD = 104
REP = 16

_info = plsc.get_sparse_core_info()
NC, NS, L = _info.num_cores, _info.num_subcores, _info.num_lanes
NW = NC * NS
BW = B // NW
ST = S // 8
KG = BW // L


def _sc_body(x_hbm, tab_hbm, out_hbm, tab_v, x_v0, x_v1, out_v0, out_v1,
             sem0, sem1, semx0, semx1):
    wid = lax.axis_index("s") * NC + lax.axis_index("c")
    b0 = wid * BW

    pltpu.sync_copy(tab_hbm, tab_v)
    iota = lax.iota(jnp.int32, L)
    zeros = jnp.full((L,), 0.0, jnp.float32)
    for d in range(D):
        tab_v[pl.ds(d * VPAD * REP, L)] = zeros

    dbase = [jnp.full((L,), d * VPAD * REP, jnp.int32) for d in range(D)]

    def x_slice(st):
        return x_hbm.at[pl.ds(st * 8, 8), pl.ds(b0, BW)]

    def out_slice(st):
        return out_hbm.at[:, pl.ds(st * 8, 8), pl.ds(b0, BW)]

    def compute_tile(x_v, out_v):
        @plsc.parallel_loop(0, 8 * KG, unroll=2)
        def k_body(k):
            s_in = k >> 5
            kk = k & (KG - 1)
            x16 = x_v[s_in, pl.ds(kk * L, L)]
            lane_idx = (x16 << 4) + iota
            vals = [plsc.load_gather(tab_v, [dbase[d] + lane_idx])
                    for d in range(D)]
            for d in range(D):
                out_v[d, s_in, pl.ds(kk * L, L)] = vals[d]

    pltpu.async_copy(x_slice(0), x_v0, semx0)
    pltpu.async_copy(x_slice(1), x_v1, semx1)

    pltpu.make_async_copy(x_slice(0), x_v0, semx0).wait()
    compute_tile(x_v0, out_v0)
    pltpu.async_copy(out_v0, out_slice(0), sem0)
    pltpu.async_copy(x_slice(2), x_v0, semx0)

    pltpu.make_async_copy(x_slice(1), x_v1, semx1).wait()
    compute_tile(x_v1, out_v1)
    pltpu.async_copy(out_v1, out_slice(1), sem1)
    pltpu.async_copy(x_slice(3), x_v1, semx1)

    def pair_body(g, carry):
        st = 2 + 2 * g
        pltpu.make_async_copy(out_v0, out_slice(st), sem0).wait()
        pltpu.make_async_copy(x_slice(st), x_v0, semx0).wait()
        compute_tile(x_v0, out_v0)
        pltpu.async_copy(out_v0, out_slice(st), sem0)
        pltpu.async_copy(x_slice(st + 2), x_v0, semx0)

        pltpu.make_async_copy(out_v1, out_slice(st + 1), sem1).wait()
        pltpu.make_async_copy(x_slice(st + 1), x_v1, semx1).wait()
        compute_tile(x_v1, out_v1)
        pltpu.async_copy(out_v1, out_slice(st + 1), sem1)

        @pl.when(st + 3 < ST)
        def _():
            pltpu.async_copy(x_slice(st + 3), x_v1, semx1)

        return carry

    lax.fori_loop(0, (ST - 3) // 2, pair_body, 0)

    pltpu.make_async_copy(out_v0, out_slice(ST - 1), sem0).wait()
    pltpu.make_async_copy(x_slice(ST - 1), x_v0, semx0).wait()
    compute_tile(x_v0, out_v0)
    pltpu.async_copy(out_v0, out_slice(ST - 1), sem0)
    pltpu.make_async_copy(out_v0, out_slice(ST - 1), sem0).wait()
    pltpu.make_async_copy(out_v1, out_slice(ST - 2), sem1).wait()


@jax.jit
def kernel(x, table):
    tab_t = jnp.zeros((D, VPAD), jnp.float32).at[:, :V].set(table.T)
    tab_rep = jnp.broadcast_to(tab_t[:, :, None], (D, VPAD, REP)).reshape(-1)
    xt = x.T.astype(jnp.int32)

    mesh = plsc.VectorSubcoreMesh(core_axis_name="c", subcore_axis_name="s")
    run = functools.partial(
        pl.kernel,
        mesh=mesh,
        out_type=jax.ShapeDtypeStruct((D, S, B), jnp.float32),
        scratch_types=[
            pltpu.VMEM((D * VPAD * REP,), jnp.float32),
            pltpu.VMEM((8, BW), jnp.int32),
            pltpu.VMEM((8, BW), jnp.int32),
            pltpu.VMEM((D, 8, BW), jnp.float32),
            pltpu.VMEM((D, 8, BW), jnp.float32),
            pltpu.SemaphoreType.DMA,
            pltpu.SemaphoreType.DMA,
            pltpu.SemaphoreType.DMA,
            pltpu.SemaphoreType.DMA,
        ],
        compiler_params=pltpu.CompilerParams(
            needs_layout_passes=False,
            use_tc_tiling_on_sc=True,
        ),
    )(_sc_body)
    out_t = run(xt, tab_rep)
    return out_t.transpose(2, 1, 0)
